# Initial kernel scaffold; baseline (speedup 1.0000x reference)
import jax
import jax.numpy as jnp
from jax.experimental import pallas as pl


def kernel(x, edge_index, W1, b1, W2, b2):
    raise NotImplementedError("write your pallas kernel here")


# speedup vs baseline: 2.6679x; 2.6679x over previous
import functools

import jax
import jax.numpy as jnp
from jax.experimental import pallas as pl
from jax.experimental.pallas import tpu as pltpu

N_NODES = 10000
D_IN = 128
D_HID = 32


def _mm_body(x_ref, w_ref, o_ref):
    o_ref[...] = jnp.dot(x_ref[...], w_ref[...],
                         preferred_element_type=jnp.float32)


def _pallas_mm(x, w):
    m, k = x.shape
    _, n = w.shape
    return pl.pallas_call(
        _mm_body,
        out_shape=jax.ShapeDtypeStruct((m, n), jnp.float32),
    )(x, w)


def _gcn(x, src, dst, dinv, W, b):
    h = _pallas_mm(x, W)
    hs = h * dinv[:, None]
    msg = jnp.take(hs, src, axis=0)
    agg = jnp.zeros_like(hs).at[dst].add(msg) + hs
    return agg * dinv[:, None] + b


def kernel(x, edge_index, W1, b1, W2, b2):
    ei = edge_index.astype(jnp.int32)
    src, dst = ei[0], ei[1]
    deg = jnp.zeros((N_NODES,), jnp.float32).at[dst].add(1.0) + 1.0
    dinv = jax.lax.rsqrt(deg)
    h = jax.nn.relu(_gcn(x, src, dst, dinv, W1, b1))
    return _gcn(h, src, dst, dinv, W2, b2)


# speedup vs baseline: 27.0670x; 10.1455x over previous
import functools

import jax
import jax.numpy as jnp
from jax import lax
from jax.experimental import pallas as pl
from jax.experimental.pallas import tpu as pltpu
from jax.experimental.pallas import tpu_sc as plsc

N = 10000
E = 320000
D_IN = 128
D_HID = 32
D_OUT = 2
D_OUTP = 8

NC, NS = 2, 16
NW = NC * NS
IW = 128
ROWS_W = 80
E_PAD = NW * ROWS_W * IW
N_PAD = 10240
STRIPE = N_PAD // NS


def _sc_mesh():
    return plsc.VectorSubcoreMesh(core_axis_name="c", subcore_axis_name="s",
                                  num_cores=NC, num_subcores=NS)


_SC_PARAMS = pltpu.CompilerParams(use_tc_tiling_on_sc=False)


def _deg_partials(ei3, zeros1, ones):
    @functools.partial(
        pl.kernel,
        out_type=(jax.ShapeDtypeStruct((N_PAD,), jnp.float32),
                  jax.ShapeDtypeStruct((N_PAD,), jnp.float32)),
        mesh=_sc_mesh(),
        scratch_types=[
            pltpu.VMEM_SHARED((N_PAD,), jnp.float32),
            pltpu.VMEM((ROWS_W, IW), jnp.int32),
            pltpu.VMEM((IW,), jnp.float32),
        ],
        compiler_params=_SC_PARAMS,
    )
    def k(ei_ref, z_ref, ones_ref, out0_ref, out1_ref, acc, didx, ones_v):
        cid = lax.axis_index("c")
        sid = lax.axis_index("s")
        wid = sid * NC + cid
        s0 = sid * STRIPE

        @pl.when(sid == 0)
        def _():
            pltpu.sync_copy(z_ref, acc)

        pltpu.sync_copy(ones_ref, ones_v)
        pltpu.sync_copy(ei_ref.at[1, pl.ds(wid * ROWS_W, ROWS_W)], didx)
        plsc.subcore_barrier()

        def body(c, carry):
            pltpu.sync_copy(ones_v, acc.at[didx.at[c]], add=True)
            return carry

        lax.fori_loop(0, ROWS_W, body, 0)
        plsc.subcore_barrier()

        @pl.when(cid == 0)
        def _():
            pltpu.sync_copy(acc.at[pl.ds(s0, STRIPE)],
                            out0_ref.at[pl.ds(s0, STRIPE)])

        @pl.when(cid == 1)
        def _():
            pltpu.sync_copy(acc.at[pl.ds(s0, STRIPE)],
                            out1_ref.at[pl.ds(s0, STRIPE)])

    return k(ei3, zeros1, ones)


def _seg_sum_partials(table, ei3, zeros2):
    d = table.shape[1]

    @functools.partial(
        pl.kernel,
        out_type=jax.ShapeDtypeStruct((2, N_PAD, d), jnp.float32),
        mesh=_sc_mesh(),
        scratch_types=[
            pltpu.VMEM_SHARED((N_PAD, d), jnp.float32),
            pltpu.VMEM((ROWS_W, IW), jnp.int32),
            pltpu.VMEM((ROWS_W, IW), jnp.int32),
            pltpu.VMEM((IW, d), jnp.float32),
            pltpu.SemaphoreType.DMA,
        ],
        compiler_params=_SC_PARAMS,
    )
    def k(t_ref, ei_ref, z_ref, out_ref, acc, sidx, didx, rows, sem):
        cid = lax.axis_index("c")
        sid = lax.axis_index("s")
        wid = sid * NC + cid
        s0 = sid * STRIPE

        pltpu.sync_copy(z_ref.at[pl.ds(s0, STRIPE)], acc.at[pl.ds(s0, STRIPE)])
        pltpu.sync_copy(ei_ref.at[0, pl.ds(wid * ROWS_W, ROWS_W)], sidx)
        pltpu.sync_copy(ei_ref.at[1, pl.ds(wid * ROWS_W, ROWS_W)], didx)
        plsc.subcore_barrier()

        def body(c, carry):
            pltpu.async_copy(t_ref.at[sidx.at[c]], rows, sem).wait()
            pltpu.sync_copy(rows, acc.at[didx.at[c]], add=True)
            return carry

        lax.fori_loop(0, ROWS_W, body, 0)
        plsc.subcore_barrier()

        @pl.when(cid == 0)
        def _():
            pltpu.sync_copy(acc.at[pl.ds(s0, STRIPE)],
                            out_ref.at[0, pl.ds(s0, STRIPE)])

        @pl.when(cid == 1)
        def _():
            pltpu.sync_copy(acc.at[pl.ds(s0, STRIPE)],
                            out_ref.at[1, pl.ds(s0, STRIPE)])

    return k(table, ei3, zeros2)


def _dinv(d0, d1):
    return lax.rsqrt(d0 + d1 + 1.0)


def _tc_layer1(degc0, degc1, x, W1):
    def body(d0_ref, d1_ref, x_ref, w_ref, o_ref):
        dinv = _dinv(d0_ref[...], d1_ref[...])
        h = jnp.dot(x_ref[...], w_ref[...], preferred_element_type=jnp.float32)
        o_ref[pl.ds(0, N), :] = h * dinv[:N]
        o_ref[pl.ds(N, N_PAD - N), :] = jnp.zeros((N_PAD - N, D_HID), jnp.float32)

    return pl.pallas_call(
        body, out_shape=jax.ShapeDtypeStruct((N_PAD, D_HID), jnp.float32),
    )(degc0, degc1, x, W1)


def _tc_layer2(degc0, degc1, part1, hs, b1, W2p):
    def body(d0_ref, d1_ref, p_ref, hs_ref, b1_ref, w2_ref, o_ref):
        dinv = _dinv(d0_ref[...], d1_ref[...])
        p = p_ref[...]
        out1 = (p[0] + p[1] + hs_ref[...]) * dinv + b1_ref[...]
        r = jnp.maximum(out1, 0.0)
        h2 = jnp.dot(r, w2_ref[...], preferred_element_type=jnp.float32)
        o_ref[pl.ds(0, N), :] = (h2 * dinv)[:N]
        o_ref[pl.ds(N, N_PAD - N), :] = jnp.zeros((N_PAD - N, D_OUTP), jnp.float32)

    return pl.pallas_call(
        body, out_shape=jax.ShapeDtypeStruct((N_PAD, D_OUTP), jnp.float32),
    )(degc0, degc1, part1, hs, b1, W2p)


def _tc_final(degc0, degc1, part2, hs2, b2p):
    def body(d0_ref, d1_ref, p_ref, hs2_ref, b2_ref, o_ref):
        dinv = _dinv(d0_ref[...], d1_ref[...])
        p = p_ref[...]
        o_ref[...] = (p[0] + p[1] + hs2_ref[...]) * dinv + b2_ref[...]

    return pl.pallas_call(
        body, out_shape=jax.ShapeDtypeStruct((N_PAD, D_OUTP), jnp.float32),
    )(degc0, degc1, part2, hs2, b2p)


def kernel(x, edge_index, W1, b1, W2, b2):
    ei = edge_index.astype(jnp.int32)
    pad = jnp.full((2, E_PAD - E), N, jnp.int32)
    ei3 = jnp.concatenate([ei, pad], axis=1).reshape(2, NW * ROWS_W, IW)

    zeros1 = jnp.zeros((N_PAD,), jnp.float32)
    zeros_h = jnp.zeros((N_PAD, D_HID), jnp.float32)
    zeros_o = jnp.zeros((N_PAD, D_OUTP), jnp.float32)
    ones = jnp.ones((IW,), jnp.float32)

    deg0, deg1 = _deg_partials(ei3, zeros1, ones)
    degc0 = deg0.reshape(N_PAD, 1)
    degc1 = deg1.reshape(N_PAD, 1)

    hs = _tc_layer1(degc0, degc1, x, W1)
    part1 = _seg_sum_partials(hs, ei3, zeros_h)

    W2p = jnp.pad(W2, ((0, 0), (0, D_OUTP - D_OUT)))
    hs2 = _tc_layer2(degc0, degc1, part1, hs, b1.reshape(1, D_HID), W2p)
    part2 = _seg_sum_partials(hs2, ei3, zeros_o)

    outp = _tc_final(degc0, degc1, part2, hs2,
                     jnp.pad(b2, (0, D_OUTP - D_OUT)).reshape(1, D_OUTP))
    return outp[:N, :D_OUT]


# speedup vs baseline: 39.0250x; 1.4418x over previous
import functools

import jax
import jax.numpy as jnp
from jax import lax
from jax.experimental import pallas as pl
from jax.experimental.pallas import tpu as pltpu
from jax.experimental.pallas import tpu_sc as plsc

N = 10000
E = 320000
D_IN = 128
D_HID = 32
D_OUT = 2
D_OUTP = 8

NC, NS = 2, 16
NW = NC * NS
IW = 128
ROWS_W = 80
E_PAD = NW * ROWS_W * IW
N_PAD = 10240
STRIPE = N_PAD // NS


def _sc_mesh():
    return plsc.VectorSubcoreMesh(core_axis_name="c", subcore_axis_name="s",
                                  num_cores=NC, num_subcores=NS)


_SC_PARAMS = pltpu.CompilerParams(use_tc_tiling_on_sc=False)


def _deg_partials(ei3, zeros1, ones):
    @functools.partial(
        pl.kernel,
        out_type=(jax.ShapeDtypeStruct((N_PAD,), jnp.float32),
                  jax.ShapeDtypeStruct((N_PAD,), jnp.float32)),
        mesh=_sc_mesh(),
        scratch_types=[
            pltpu.VMEM_SHARED((N_PAD,), jnp.float32),
            pltpu.VMEM((ROWS_W, IW), jnp.int32),
            pltpu.VMEM((IW,), jnp.float32),
        ],
        compiler_params=_SC_PARAMS,
    )
    def k(ei_ref, z_ref, ones_ref, out0_ref, out1_ref, acc, didx, ones_v):
        cid = lax.axis_index("c")
        sid = lax.axis_index("s")
        wid = sid * NC + cid
        s0 = sid * STRIPE

        pltpu.sync_copy(z_ref.at[pl.ds(s0, STRIPE)], acc.at[pl.ds(s0, STRIPE)])
        pltpu.sync_copy(ones_ref, ones_v)
        pltpu.sync_copy(ei_ref.at[1, pl.ds(wid * ROWS_W, ROWS_W)], didx)
        plsc.subcore_barrier()

        def body(c, carry):
            pltpu.sync_copy(ones_v, acc.at[didx.at[c]], add=True)
            return carry

        lax.fori_loop(0, ROWS_W, body, 0)
        plsc.subcore_barrier()

        @pl.when(cid == 0)
        def _():
            pltpu.sync_copy(acc.at[pl.ds(s0, STRIPE)],
                            out0_ref.at[pl.ds(s0, STRIPE)])

        @pl.when(cid == 1)
        def _():
            pltpu.sync_copy(acc.at[pl.ds(s0, STRIPE)],
                            out1_ref.at[pl.ds(s0, STRIPE)])

    return k(ei3, zeros1, ones)


def _seg_sum_partials(table, ei3, zeros2):
    d = table.shape[1]

    @functools.partial(
        pl.kernel,
        out_type=jax.ShapeDtypeStruct((2, N_PAD, d), jnp.float32),
        mesh=_sc_mesh(),
        scratch_types=[
            pltpu.VMEM_SHARED((N_PAD, d), jnp.float32),
            pltpu.VMEM((ROWS_W + 1, IW), jnp.int32),
            pltpu.VMEM((ROWS_W, IW), jnp.int32),
            pltpu.VMEM((IW, d), jnp.float32),
            pltpu.VMEM((IW, d), jnp.float32),
            pltpu.SemaphoreType.DMA,
            pltpu.SemaphoreType.DMA,
        ],
        compiler_params=_SC_PARAMS,
    )
    def k(t_ref, ei_ref, z_ref, out_ref, acc, sidx, didx, rows0, rows1,
          sem0, sem1):
        cid = lax.axis_index("c")
        sid = lax.axis_index("s")
        wid = sid * NC + cid
        s0 = sid * STRIPE

        pltpu.sync_copy(z_ref.at[pl.ds(s0, STRIPE)], acc.at[pl.ds(s0, STRIPE)])
        pltpu.sync_copy(ei_ref.at[0, pl.ds(wid * ROWS_W, ROWS_W)],
                        sidx.at[pl.ds(0, ROWS_W)])
        pltpu.sync_copy(ei_ref.at[0, pl.ds(wid * ROWS_W, 1)],
                        sidx.at[pl.ds(ROWS_W, 1)])
        pltpu.sync_copy(ei_ref.at[1, pl.ds(wid * ROWS_W, ROWS_W)], didx)
        plsc.subcore_barrier()

        pltpu.async_copy(t_ref.at[sidx.at[0]], rows0, sem0).wait()

        def body(c2, carry):
            r0 = 2 * c2
            d1 = pltpu.async_copy(t_ref.at[sidx.at[r0 + 1]], rows1, sem1)
            pltpu.sync_copy(rows0, acc.at[didx.at[r0]], add=True)
            d1.wait()
            d0 = pltpu.async_copy(t_ref.at[sidx.at[r0 + 2]], rows0, sem0)
            pltpu.sync_copy(rows1, acc.at[didx.at[r0 + 1]], add=True)
            d0.wait()
            return carry

        lax.fori_loop(0, ROWS_W // 2, body, 0)
        plsc.subcore_barrier()

        @pl.when(cid == 0)
        def _():
            pltpu.sync_copy(acc.at[pl.ds(s0, STRIPE)],
                            out_ref.at[0, pl.ds(s0, STRIPE)])

        @pl.when(cid == 1)
        def _():
            pltpu.sync_copy(acc.at[pl.ds(s0, STRIPE)],
                            out_ref.at[1, pl.ds(s0, STRIPE)])

    return k(table, ei3, zeros2)


def _dinv(d0, d1):
    return lax.rsqrt(d0 + d1 + 1.0)


def _tc_layer1(degc0, degc1, x, W1):
    def body(d0_ref, d1_ref, x_ref, w_ref, o_ref):
        dinv = _dinv(d0_ref[...], d1_ref[...])
        h = jnp.dot(x_ref[...], w_ref[...], preferred_element_type=jnp.float32)
        o_ref[pl.ds(0, N), :] = h * dinv[:N]
        o_ref[pl.ds(N, N_PAD - N), :] = jnp.zeros((N_PAD - N, D_HID), jnp.float32)

    return pl.pallas_call(
        body, out_shape=jax.ShapeDtypeStruct((N_PAD, D_HID), jnp.float32),
    )(degc0, degc1, x, W1)


def _tc_layer2(degc0, degc1, part1, hs, b1, W2p):
    def body(d0_ref, d1_ref, p_ref, hs_ref, b1_ref, w2_ref, o_ref):
        dinv = _dinv(d0_ref[...], d1_ref[...])
        p = p_ref[...]
        out1 = (p[0] + p[1] + hs_ref[...]) * dinv + b1_ref[...]
        r = jnp.maximum(out1, 0.0)
        h2 = jnp.dot(r, w2_ref[...], preferred_element_type=jnp.float32)
        o_ref[pl.ds(0, N), :] = (h2 * dinv)[:N]
        o_ref[pl.ds(N, N_PAD - N), :] = jnp.zeros((N_PAD - N, D_OUTP), jnp.float32)

    return pl.pallas_call(
        body, out_shape=jax.ShapeDtypeStruct((N_PAD, D_OUTP), jnp.float32),
    )(degc0, degc1, part1, hs, b1, W2p)


def _tc_final(degc0, degc1, part2, hs2, b2p):
    def body(d0_ref, d1_ref, p_ref, hs2_ref, b2_ref, o_ref):
        dinv = _dinv(d0_ref[...], d1_ref[...])
        p = p_ref[...]
        o_ref[...] = (p[0] + p[1] + hs2_ref[...]) * dinv + b2_ref[...]

    return pl.pallas_call(
        body, out_shape=jax.ShapeDtypeStruct((N_PAD, D_OUTP), jnp.float32),
    )(degc0, degc1, part2, hs2, b2p)


def kernel(x, edge_index, W1, b1, W2, b2):
    ei = edge_index.astype(jnp.int32)
    pad_idx = N + jnp.arange(E_PAD - E, dtype=jnp.int32) % (N_PAD - N)
    ei3 = jnp.concatenate(
        [ei, jnp.stack([pad_idx, pad_idx])], axis=1).reshape(2, NW * ROWS_W, IW)

    zeros1 = jnp.zeros((N_PAD,), jnp.float32)
    zeros_h = jnp.zeros((N_PAD, D_HID), jnp.float32)
    zeros_o = jnp.zeros((N_PAD, D_OUTP), jnp.float32)
    ones = jnp.ones((IW,), jnp.float32)

    deg0, deg1 = _deg_partials(ei3, zeros1, ones)
    degc0 = deg0.reshape(N_PAD, 1)
    degc1 = deg1.reshape(N_PAD, 1)

    hs = _tc_layer1(degc0, degc1, x, W1)
    part1 = _seg_sum_partials(hs, ei3, zeros_h)

    W2p = jnp.pad(W2, ((0, 0), (0, D_OUTP - D_OUT)))
    hs2 = _tc_layer2(degc0, degc1, part1, hs, b1.reshape(1, D_HID), W2p)
    part2 = _seg_sum_partials(hs2, ei3, zeros_o)

    outp = _tc_final(degc0, degc1, part2, hs2,
                     jnp.pad(b2, (0, D_OUTP - D_OUT)).reshape(1, D_OUTP))
    return outp[:N, :D_OUT]


# speedup vs baseline: 57.3475x; 1.4695x over previous
import functools

import jax
import jax.numpy as jnp
from jax import lax
from jax.experimental import pallas as pl
from jax.experimental.pallas import tpu as pltpu
from jax.experimental.pallas import tpu_sc as plsc

N = 10000
E = 320000
D_IN = 128
D_HID = 32
D_OUT = 2
D_OUTP = 8

NC, NS = 2, 16
NW = NC * NS
IW = 128
ROWS_W = 80
EW = ROWS_W * IW
GIW = 1024
NG = EW // GIW
E_PAD = NW * ROWS_W * IW
N_PAD = 10240
STRIPE = N_PAD // NS


def _sc_mesh():
    return plsc.VectorSubcoreMesh(core_axis_name="c", subcore_axis_name="s",
                                  num_cores=NC, num_subcores=NS)


_SC_PARAMS = pltpu.CompilerParams(use_tc_tiling_on_sc=False)


def _deg_partials(ei3, zeros1, ones):
    @functools.partial(
        pl.kernel,
        out_type=(jax.ShapeDtypeStruct((N_PAD,), jnp.float32),
                  jax.ShapeDtypeStruct((N_PAD,), jnp.float32)),
        mesh=_sc_mesh(),
        scratch_types=[
            pltpu.VMEM_SHARED((N_PAD,), jnp.float32),
            pltpu.VMEM((EW,), jnp.int32),
            pltpu.VMEM((GIW,), jnp.float32),
        ],
        compiler_params=_SC_PARAMS,
    )
    def k(ei_ref, z_ref, ones_ref, out0_ref, out1_ref, acc, didx, ones_v):
        cid = lax.axis_index("c")
        sid = lax.axis_index("s")
        wid = sid * NC + cid
        s0 = sid * STRIPE

        pltpu.sync_copy(z_ref.at[pl.ds(s0, STRIPE)], acc.at[pl.ds(s0, STRIPE)])
        pltpu.sync_copy(ones_ref, ones_v)
        pltpu.sync_copy(ei_ref.at[1, pl.ds(wid * EW, EW)], didx)
        plsc.subcore_barrier()

        def body(c, carry):
            pltpu.sync_copy(ones_v, acc.at[didx.at[pl.ds(c * GIW, GIW)]],
                            add=True)
            return carry

        lax.fori_loop(0, NG, body, 0)
        plsc.subcore_barrier()

        @pl.when(cid == 0)
        def _():
            pltpu.sync_copy(acc.at[pl.ds(s0, STRIPE)],
                            out0_ref.at[pl.ds(s0, STRIPE)])

        @pl.when(cid == 1)
        def _():
            pltpu.sync_copy(acc.at[pl.ds(s0, STRIPE)],
                            out1_ref.at[pl.ds(s0, STRIPE)])

    return k(ei3, zeros1, ones)


def _seg_sum_partials(table, ei3, zeros2):
    d = table.shape[1]

    @functools.partial(
        pl.kernel,
        out_type=jax.ShapeDtypeStruct((2, N_PAD, d), jnp.float32),
        mesh=_sc_mesh(),
        scratch_types=[
            pltpu.VMEM_SHARED((N_PAD, d), jnp.float32),
            pltpu.VMEM((EW + GIW,), jnp.int32),
            pltpu.VMEM((EW,), jnp.int32),
            pltpu.VMEM((GIW, d), jnp.float32),
            pltpu.VMEM((GIW, d), jnp.float32),
            pltpu.SemaphoreType.DMA,
            pltpu.SemaphoreType.DMA,
        ],
        compiler_params=_SC_PARAMS,
    )
    def k(t_ref, ei_ref, z_ref, out_ref, acc, sidx, didx, rows0, rows1,
          sem0, sem1):
        cid = lax.axis_index("c")
        sid = lax.axis_index("s")
        wid = sid * NC + cid
        s0 = sid * STRIPE

        pltpu.sync_copy(z_ref.at[pl.ds(s0, STRIPE)], acc.at[pl.ds(s0, STRIPE)])
        pltpu.sync_copy(ei_ref.at[0, pl.ds(wid * EW, EW)],
                        sidx.at[pl.ds(0, EW)])
        pltpu.sync_copy(ei_ref.at[0, pl.ds(wid * EW, GIW)],
                        sidx.at[pl.ds(EW, GIW)])
        pltpu.sync_copy(ei_ref.at[1, pl.ds(wid * EW, EW)], didx)
        plsc.subcore_barrier()

        pltpu.async_copy(t_ref.at[sidx.at[pl.ds(0, GIW)]], rows0, sem0).wait()

        def body(c2, carry):
            g0 = 2 * c2
            d1 = pltpu.async_copy(
                t_ref.at[sidx.at[pl.ds((g0 + 1) * GIW, GIW)]], rows1, sem1)
            pltpu.sync_copy(rows0, acc.at[didx.at[pl.ds(g0 * GIW, GIW)]],
                            add=True)
            d1.wait()
            d0 = pltpu.async_copy(
                t_ref.at[sidx.at[pl.ds((g0 + 2) * GIW, GIW)]], rows0, sem0)
            pltpu.sync_copy(rows1, acc.at[didx.at[pl.ds((g0 + 1) * GIW, GIW)]],
                            add=True)
            d0.wait()
            return carry

        lax.fori_loop(0, NG // 2, body, 0)
        plsc.subcore_barrier()

        @pl.when(cid == 0)
        def _():
            pltpu.sync_copy(acc.at[pl.ds(s0, STRIPE)],
                            out_ref.at[0, pl.ds(s0, STRIPE)])

        @pl.when(cid == 1)
        def _():
            pltpu.sync_copy(acc.at[pl.ds(s0, STRIPE)],
                            out_ref.at[1, pl.ds(s0, STRIPE)])

    return k(table, ei3, zeros2)


def _dinv(d0, d1):
    return lax.rsqrt(d0 + d1 + 1.0)


def _tc_layer1(degc0, degc1, x, W1):
    def body(d0_ref, d1_ref, x_ref, w_ref, o_ref):
        dinv = _dinv(d0_ref[...], d1_ref[...])
        h = jnp.dot(x_ref[...], w_ref[...], preferred_element_type=jnp.float32)
        o_ref[pl.ds(0, N), :] = h * dinv[:N]
        o_ref[pl.ds(N, N_PAD - N), :] = jnp.zeros((N_PAD - N, D_HID), jnp.float32)

    return pl.pallas_call(
        body, out_shape=jax.ShapeDtypeStruct((N_PAD, D_HID), jnp.float32),
    )(degc0, degc1, x, W1)


def _tc_layer2(degc0, degc1, part1, hs, b1, W2p):
    def body(d0_ref, d1_ref, p_ref, hs_ref, b1_ref, w2_ref, o_ref):
        dinv = _dinv(d0_ref[...], d1_ref[...])
        p = p_ref[...]
        out1 = (p[0] + p[1] + hs_ref[...]) * dinv + b1_ref[...]
        r = jnp.maximum(out1, 0.0)
        h2 = jnp.dot(r, w2_ref[...], preferred_element_type=jnp.float32)
        o_ref[pl.ds(0, N), :] = (h2 * dinv)[:N]
        o_ref[pl.ds(N, N_PAD - N), :] = jnp.zeros((N_PAD - N, D_OUTP), jnp.float32)

    return pl.pallas_call(
        body, out_shape=jax.ShapeDtypeStruct((N_PAD, D_OUTP), jnp.float32),
    )(degc0, degc1, part1, hs, b1, W2p)


def _tc_final(degc0, degc1, part2, hs2, b2p):
    def body(d0_ref, d1_ref, p_ref, hs2_ref, b2_ref, o_ref):
        dinv = _dinv(d0_ref[...], d1_ref[...])
        p = p_ref[...]
        o_ref[...] = (p[0] + p[1] + hs2_ref[...]) * dinv + b2_ref[...]

    return pl.pallas_call(
        body, out_shape=jax.ShapeDtypeStruct((N_PAD, D_OUTP), jnp.float32),
    )(degc0, degc1, part2, hs2, b2p)


def kernel(x, edge_index, W1, b1, W2, b2):
    ei = edge_index.astype(jnp.int32)
    pad_idx = N + jnp.arange(E_PAD - E, dtype=jnp.int32) % (N_PAD - N)
    ei3 = jnp.concatenate([ei, jnp.stack([pad_idx, pad_idx])], axis=1)

    zeros1 = jnp.zeros((N_PAD,), jnp.float32)
    zeros_h = jnp.zeros((N_PAD, D_HID), jnp.float32)
    zeros_o = jnp.zeros((N_PAD, D_OUTP), jnp.float32)
    ones = jnp.ones((GIW,), jnp.float32)

    deg0, deg1 = _deg_partials(ei3, zeros1, ones)
    degc0 = deg0.reshape(N_PAD, 1)
    degc1 = deg1.reshape(N_PAD, 1)

    hs = _tc_layer1(degc0, degc1, x, W1)
    part1 = _seg_sum_partials(hs, ei3, zeros_h)

    W2p = jnp.pad(W2, ((0, 0), (0, D_OUTP - D_OUT)))
    hs2 = _tc_layer2(degc0, degc1, part1, hs, b1.reshape(1, D_HID), W2p)
    part2 = _seg_sum_partials(hs2, ei3, zeros_o)

    outp = _tc_final(degc0, degc1, part2, hs2,
                     jnp.pad(b2, (0, D_OUTP - D_OUT)).reshape(1, D_OUTP))
    return outp[:N, :D_OUT]


# speedup vs baseline: 60.8550x; 1.0612x over previous
import functools

import jax
import jax.numpy as jnp
from jax import lax
from jax.experimental import pallas as pl
from jax.experimental.pallas import tpu as pltpu
from jax.experimental.pallas import tpu_sc as plsc

N = 10000
E = 320000
D_IN = 128
D_HID = 32
D_OUT = 2
D_OUTP = 8

NC, NS = 2, 16
NW = NC * NS
EW = E // NW
NG = 10
GIW = EW // NG
N_PAD = 10240
STRIPE = N_PAD // NS


def _sc_mesh():
    return plsc.VectorSubcoreMesh(core_axis_name="c", subcore_axis_name="s",
                                  num_cores=NC, num_subcores=NS)


_SC_PARAMS = pltpu.CompilerParams(use_tc_tiling_on_sc=False)


def _deg_partials(ei3, zeros1, ones):
    @functools.partial(
        pl.kernel,
        out_type=(jax.ShapeDtypeStruct((N_PAD,), jnp.float32),
                  jax.ShapeDtypeStruct((N_PAD,), jnp.float32)),
        mesh=_sc_mesh(),
        scratch_types=[
            pltpu.VMEM_SHARED((N_PAD,), jnp.float32),
            pltpu.VMEM((EW,), jnp.int32),
            pltpu.VMEM((GIW,), jnp.float32),
        ],
        compiler_params=_SC_PARAMS,
    )
    def k(ei_ref, z_ref, ones_ref, out0_ref, out1_ref, acc, didx, ones_v):
        cid = lax.axis_index("c")
        sid = lax.axis_index("s")
        wid = sid * NC + cid
        s0 = sid * STRIPE

        pltpu.sync_copy(z_ref.at[pl.ds(s0, STRIPE)], acc.at[pl.ds(s0, STRIPE)])
        pltpu.sync_copy(ones_ref, ones_v)
        pltpu.sync_copy(ei_ref.at[1, pl.ds(wid * EW, EW)], didx)
        plsc.subcore_barrier()

        def body(c, carry):
            pltpu.sync_copy(ones_v, acc.at[didx.at[pl.ds(c * GIW, GIW)]],
                            add=True)
            return carry

        lax.fori_loop(0, NG, body, 0)
        plsc.subcore_barrier()

        @pl.when(cid == 0)
        def _():
            pltpu.sync_copy(acc.at[pl.ds(s0, STRIPE)],
                            out0_ref.at[pl.ds(s0, STRIPE)])

        @pl.when(cid == 1)
        def _():
            pltpu.sync_copy(acc.at[pl.ds(s0, STRIPE)],
                            out1_ref.at[pl.ds(s0, STRIPE)])

    return k(ei3, zeros1, ones)


def _seg_sum_partials(table, ei3, zeros2):
    d = table.shape[1]

    @functools.partial(
        pl.kernel,
        out_type=jax.ShapeDtypeStruct((2, N_PAD, d), jnp.float32),
        mesh=_sc_mesh(),
        scratch_types=[
            pltpu.VMEM_SHARED((N_PAD, d), jnp.float32),
            pltpu.VMEM((EW,), jnp.int32),
            pltpu.VMEM((GIW,), jnp.int32),
            pltpu.VMEM((GIW,), jnp.int32),
            pltpu.VMEM((GIW, d), jnp.float32),
            pltpu.VMEM((GIW, d), jnp.float32),
            pltpu.VMEM((GIW, d), jnp.float32),
            pltpu.SemaphoreType.DMA,
            pltpu.SemaphoreType.DMA,
            pltpu.SemaphoreType.DMA,
            pltpu.SemaphoreType.DMA,
            pltpu.SemaphoreType.DMA,
        ],
        compiler_params=_SC_PARAMS,
    )
    def k(t_ref, ei_ref, z_ref, out_ref, acc, sidx, didx0, didx1,
          rows0, rows1, rows2, sem0, sem1, sem2, semd0, semd1):
        cid = lax.axis_index("c")
        sid = lax.axis_index("s")
        wid = sid * NC + cid
        s0 = sid * STRIPE

        pltpu.sync_copy(z_ref.at[pl.ds(s0, STRIPE)], acc.at[pl.ds(s0, STRIPE)])
        pltpu.sync_copy(ei_ref.at[0, pl.ds(wid * EW, EW)], sidx)
        plsc.subcore_barrier()

        bufs = (rows0, rows1, rows2)
        sems = (sem0, sem1, sem2)
        dbufs = (didx0, didx1)
        dsems = (semd0, semd1)

        def gather(g):
            return pltpu.async_copy(
                t_ref.at[sidx.at[pl.ds(g * GIW, GIW)]], bufs[g % 3],
                sems[g % 3])

        def dload(g):
            return pltpu.async_copy(
                ei_ref.at[1, pl.ds(wid * EW + g * GIW, GIW)], dbufs[g % 2],
                dsems[g % 2])

        gd = {0: gather(0), 1: gather(1)}
        dd = {0: dload(0), 1: dload(1)}
        for g in range(NG):
            dd[g].wait()
            gd[g].wait()
            if g + 2 < NG:
                gd[g + 2] = gather(g + 2)
            pltpu.sync_copy(bufs[g % 3], acc.at[dbufs[g % 2]], add=True)
            if g + 2 < NG:
                dd[g + 2] = dload(g + 2)
        plsc.subcore_barrier()

        @pl.when(cid == 0)
        def _():
            pltpu.sync_copy(acc.at[pl.ds(s0, STRIPE)],
                            out_ref.at[0, pl.ds(s0, STRIPE)])

        @pl.when(cid == 1)
        def _():
            pltpu.sync_copy(acc.at[pl.ds(s0, STRIPE)],
                            out_ref.at[1, pl.ds(s0, STRIPE)])

    return k(table, ei3, zeros2)


def _dinv(d0, d1):
    return lax.rsqrt(d0 + d1 + 1.0)


def _tc_matmul1(x, W1):
    def body(x_ref, w_ref, o_ref):
        o_ref[...] = jnp.dot(x_ref[...], w_ref[...],
                             preferred_element_type=jnp.float32)

    return pl.pallas_call(
        body, out_shape=jax.ShapeDtypeStruct((N, D_HID), jnp.float32),
    )(x, W1)


def _tc_scale1(degc0, degc1, h):
    def body(d0_ref, d1_ref, h_ref, o_ref):
        dinv = _dinv(d0_ref[...], d1_ref[...])
        o_ref[pl.ds(0, N), :] = h_ref[...] * dinv[:N]
        o_ref[pl.ds(N, N_PAD - N), :] = jnp.zeros((N_PAD - N, D_HID), jnp.float32)

    return pl.pallas_call(
        body, out_shape=jax.ShapeDtypeStruct((N_PAD, D_HID), jnp.float32),
    )(degc0, degc1, h)


def _tc_layer2(degc0, degc1, part1, hs, b1, W2p):
    def body(d0_ref, d1_ref, p_ref, hs_ref, b1_ref, w2_ref, o_ref):
        dinv = _dinv(d0_ref[...], d1_ref[...])
        p = p_ref[...]
        out1 = (p[0] + p[1] + hs_ref[...]) * dinv + b1_ref[...]
        r = jnp.maximum(out1, 0.0)
        h2 = jnp.dot(r, w2_ref[...], preferred_element_type=jnp.float32)
        o_ref[pl.ds(0, N), :] = (h2 * dinv)[:N]
        o_ref[pl.ds(N, N_PAD - N), :] = jnp.zeros((N_PAD - N, D_OUTP), jnp.float32)

    return pl.pallas_call(
        body, out_shape=jax.ShapeDtypeStruct((N_PAD, D_OUTP), jnp.float32),
    )(degc0, degc1, part1, hs, b1, W2p)


def _tc_final(degc0, degc1, part2, hs2, b2p):
    def body(d0_ref, d1_ref, p_ref, hs2_ref, b2_ref, o_ref):
        dinv = _dinv(d0_ref[...], d1_ref[...])
        p = p_ref[...]
        o_ref[...] = (p[0] + p[1] + hs2_ref[...]) * dinv + b2_ref[...]

    return pl.pallas_call(
        body, out_shape=jax.ShapeDtypeStruct((N_PAD, D_OUTP), jnp.float32),
    )(degc0, degc1, part2, hs2, b2p)


def kernel(x, edge_index, W1, b1, W2, b2):
    ei3 = edge_index.astype(jnp.int32)

    zeros1 = jnp.zeros((N_PAD,), jnp.float32)
    zeros_h = jnp.zeros((N_PAD, D_HID), jnp.float32)
    zeros_o = jnp.zeros((N_PAD, D_OUTP), jnp.float32)
    ones = jnp.ones((GIW,), jnp.float32)

    deg0, deg1 = _deg_partials(ei3, zeros1, ones)
    degc0 = deg0.reshape(N_PAD, 1)
    degc1 = deg1.reshape(N_PAD, 1)

    h = _tc_matmul1(x, W1)
    hs = _tc_scale1(degc0, degc1, h)
    part1 = _seg_sum_partials(hs, ei3, zeros_h)

    W2p = jnp.pad(W2, ((0, 0), (0, D_OUTP - D_OUT)))
    hs2 = _tc_layer2(degc0, degc1, part1, hs, b1.reshape(1, D_HID), W2p)
    part2 = _seg_sum_partials(hs2, ei3, zeros_o)

    outp = _tc_final(degc0, degc1, part2, hs2,
                     jnp.pad(b2, (0, D_OUTP - D_OUT)).reshape(1, D_OUTP))
    return outp[:N, :D_OUT]


# speedup vs baseline: 62.5757x; 1.0283x over previous
import functools

import jax
import jax.numpy as jnp
from jax import lax
from jax.experimental import pallas as pl
from jax.experimental.pallas import tpu as pltpu
from jax.experimental.pallas import tpu_sc as plsc

N = 10000
E = 320000
D_IN = 128
D_HID = 32
D_OUT = 2
D_OUTP = 8

NC, NS = 2, 16
NW = NC * NS
EW = E // NW
NG = 10
GIW = EW // NG
N_PAD = 10240
STRIPE = N_PAD // NS


def _sc_mesh():
    return plsc.VectorSubcoreMesh(core_axis_name="c", subcore_axis_name="s",
                                  num_cores=NC, num_subcores=NS)


_SC_PARAMS = pltpu.CompilerParams(use_tc_tiling_on_sc=False)


def _deg_partials(ei3, zeros1, ones):
    @functools.partial(
        pl.kernel,
        out_type=(jax.ShapeDtypeStruct((N_PAD,), jnp.float32),
                  jax.ShapeDtypeStruct((N_PAD,), jnp.float32)),
        mesh=_sc_mesh(),
        scratch_types=[
            pltpu.VMEM_SHARED((N_PAD,), jnp.float32),
            pltpu.VMEM((EW,), jnp.int32),
            pltpu.VMEM((GIW,), jnp.float32),
        ],
        compiler_params=_SC_PARAMS,
    )
    def k(ei_ref, z_ref, ones_ref, out0_ref, out1_ref, acc, didx, ones_v):
        cid = lax.axis_index("c")
        sid = lax.axis_index("s")
        wid = sid * NC + cid
        s0 = sid * STRIPE

        pltpu.sync_copy(z_ref.at[pl.ds(s0, STRIPE)], acc.at[pl.ds(s0, STRIPE)])
        pltpu.sync_copy(ones_ref, ones_v)
        pltpu.sync_copy(ei_ref.at[1, pl.ds(wid * EW, EW)], didx)
        plsc.subcore_barrier()

        def body(c, carry):
            pltpu.sync_copy(ones_v, acc.at[didx.at[pl.ds(c * GIW, GIW)]],
                            add=True)
            return carry

        lax.fori_loop(0, NG, body, 0)
        plsc.subcore_barrier()

        @pl.when(cid == 0)
        def _():
            pltpu.sync_copy(acc.at[pl.ds(s0, STRIPE)],
                            out0_ref.at[pl.ds(s0, STRIPE)])

        @pl.when(cid == 1)
        def _():
            pltpu.sync_copy(acc.at[pl.ds(s0, STRIPE)],
                            out1_ref.at[pl.ds(s0, STRIPE)])

    return k(ei3, zeros1, ones)


def _seg_sum_partials(table, ei3, zeros2):
    d = table.shape[1]

    @functools.partial(
        pl.kernel,
        out_type=jax.ShapeDtypeStruct((2, N_PAD, d), jnp.float32),
        mesh=_sc_mesh(),
        scratch_types=[
            pltpu.VMEM_SHARED((N_PAD, d), jnp.float32),
            pltpu.VMEM((EW,), jnp.int32),
            pltpu.VMEM((GIW,), jnp.int32),
            pltpu.VMEM((GIW,), jnp.int32),
            pltpu.VMEM((GIW, d), jnp.float32),
            pltpu.VMEM((GIW, d), jnp.float32),
            pltpu.VMEM((GIW, d), jnp.float32),
            pltpu.SemaphoreType.DMA,
            pltpu.SemaphoreType.DMA,
            pltpu.SemaphoreType.DMA,
            pltpu.SemaphoreType.DMA,
            pltpu.SemaphoreType.DMA,
        ],
        compiler_params=_SC_PARAMS,
    )
    def k(t_ref, ei_ref, z_ref, out_ref, acc, sidx, didx0, didx1,
          rows0, rows1, rows2, sem0, sem1, sem2, semd0, semd1):
        cid = lax.axis_index("c")
        sid = lax.axis_index("s")
        wid = sid * NC + cid
        s0 = sid * STRIPE

        pltpu.sync_copy(z_ref.at[pl.ds(s0, STRIPE)], acc.at[pl.ds(s0, STRIPE)])
        pltpu.sync_copy(ei_ref.at[0, pl.ds(wid * EW, EW)], sidx)
        plsc.subcore_barrier()

        bufs = (rows0, rows1, rows2)
        sems = (sem0, sem1, sem2)
        dbufs = (didx0, didx1)
        dsems = (semd0, semd1)

        def gather(g):
            return pltpu.async_copy(
                t_ref.at[sidx.at[pl.ds(g * GIW, GIW)]], bufs[g % 3],
                sems[g % 3])

        def dload(g):
            return pltpu.async_copy(
                ei_ref.at[1, pl.ds(wid * EW + g * GIW, GIW)], dbufs[g % 2],
                dsems[g % 2])

        gd = {0: gather(0), 1: gather(1)}
        dd = {0: dload(0), 1: dload(1)}
        for g in range(NG):
            dd[g].wait()
            gd[g].wait()
            if g + 2 < NG:
                gd[g + 2] = gather(g + 2)
            pltpu.sync_copy(bufs[g % 3], acc.at[dbufs[g % 2]], add=True)
            if g + 2 < NG:
                dd[g + 2] = dload(g + 2)
        plsc.subcore_barrier()

        @pl.when(cid == 0)
        def _():
            pltpu.sync_copy(acc.at[pl.ds(s0, STRIPE)],
                            out_ref.at[0, pl.ds(s0, STRIPE)])

        @pl.when(cid == 1)
        def _():
            pltpu.sync_copy(acc.at[pl.ds(s0, STRIPE)],
                            out_ref.at[1, pl.ds(s0, STRIPE)])

    return k(table, ei3, zeros2)


def _dinv(d0, d1):
    return lax.rsqrt(d0 + d1 + 1.0)


def _tc_layer1(degc0, degc1, x, W1):
    def body(d0_ref, d1_ref, x_ref, w_ref, o_ref):
        dinv = _dinv(d0_ref[...], d1_ref[...])
        h = jnp.dot(x_ref[...], w_ref[...], preferred_element_type=jnp.float32)
        o_ref[pl.ds(0, N), :] = h * dinv[:N]
        o_ref[pl.ds(N, N_PAD - N), :] = jnp.zeros((N_PAD - N, D_HID), jnp.float32)

    return pl.pallas_call(
        body, out_shape=jax.ShapeDtypeStruct((N_PAD, D_HID), jnp.float32),
    )(degc0, degc1, x, W1)


def _tc_layer2(degc0, degc1, part1, hs, b1, W2p):
    def body(d0_ref, d1_ref, p_ref, hs_ref, b1_ref, w2_ref, o_ref):
        dinv = _dinv(d0_ref[...], d1_ref[...])
        p = p_ref[...]
        out1 = (p[0] + p[1] + hs_ref[...]) * dinv + b1_ref[...]
        r = jnp.maximum(out1, 0.0)
        h2 = jnp.dot(r, w2_ref[...], preferred_element_type=jnp.float32)
        o_ref[pl.ds(0, N), :] = (h2 * dinv)[:N]
        o_ref[pl.ds(N, N_PAD - N), :] = jnp.zeros((N_PAD - N, D_OUTP), jnp.float32)

    return pl.pallas_call(
        body, out_shape=jax.ShapeDtypeStruct((N_PAD, D_OUTP), jnp.float32),
    )(degc0, degc1, part1, hs, b1, W2p)


def _tc_final(degc0, degc1, part2, hs2, b2p):
    def body(d0_ref, d1_ref, p_ref, hs2_ref, b2_ref, o_ref):
        dinv = _dinv(d0_ref[...], d1_ref[...])
        p = p_ref[...]
        o_ref[...] = (p[0] + p[1] + hs2_ref[...]) * dinv + b2_ref[...]

    return pl.pallas_call(
        body, out_shape=jax.ShapeDtypeStruct((N_PAD, D_OUTP), jnp.float32),
    )(degc0, degc1, part2, hs2, b2p)


def kernel(x, edge_index, W1, b1, W2, b2):
    ei3 = edge_index.astype(jnp.int32)

    zeros1 = jnp.zeros((N_PAD,), jnp.float32)
    zeros_h = jnp.zeros((N_PAD, D_HID), jnp.float32)
    zeros_o = jnp.zeros((N_PAD, D_OUTP), jnp.float32)
    ones = jnp.ones((GIW,), jnp.float32)

    deg0, deg1 = _deg_partials(ei3, zeros1, ones)
    degc0 = deg0.reshape(N_PAD, 1)
    degc1 = deg1.reshape(N_PAD, 1)

    hs = _tc_layer1(degc0, degc1, x, W1)
    part1 = _seg_sum_partials(hs, ei3, zeros_h)

    W2p = jnp.pad(W2, ((0, 0), (0, D_OUTP - D_OUT)))
    hs2 = _tc_layer2(degc0, degc1, part1, hs, b1.reshape(1, D_HID), W2p)
    part2 = _seg_sum_partials(hs2, ei3, zeros_o)

    outp = _tc_final(degc0, degc1, part2, hs2,
                     jnp.pad(b2, (0, D_OUTP - D_OUT)).reshape(1, D_OUTP))
    return outp[:N, :D_OUT]
